# Initial kernel scaffold; baseline (speedup 1.0000x reference)
import jax
import jax.numpy as jnp
from jax.experimental import pallas as pl


def kernel(inputs, main_table, re_lut):
    raise NotImplementedError("write your pallas kernel here")


# speedup vs baseline: 2.3742x; 2.3742x over previous
import functools

import jax
import jax.numpy as jnp
from jax import lax
from jax.experimental import pallas as pl
from jax.experimental.pallas import tpu as pltpu
from jax.experimental.pallas import tpu_sc as plsc

_L = 16


@functools.lru_cache(maxsize=None)
def _build(N, D, V, A, NC, NS):
    NW = NC * NS
    NPW = N // NW
    C = 640
    assert NPW % C == 0 and C % 128 == 0 and N % NW == 0
    NCH = NPW // C
    NB = C // 128
    mesh = plsc.VectorSubcoreMesh(
        core_axis_name="c", subcore_axis_name="s",
        num_cores=NC, num_subcores=NS)

    @functools.partial(
        pl.kernel,
        out_type=jax.ShapeDtypeStruct((N, D), jnp.float32),
        mesh=mesh,
        scratch_types=[
            pltpu.VMEM((C,), jnp.int32),
            pltpu.VMEM((C,), jnp.int32),
            pltpu.VMEM((C, D), jnp.float32),
            pltpu.VMEM((A, D), jnp.float32),
            pltpu.SemaphoreType.DMA,
        ],
        compiler_params=pltpu.CompilerParams(use_tc_tiling_on_sc=False, needs_layout_passes=False),
    )
    def k(ids_hbm, main_hbm, aux_hbm, out_hbm, raw_v, idx_v, buf_v, aux_v, sem):
        wid = lax.axis_index("s") * NC + lax.axis_index("c")
        base = wid * NPW
        pltpu.sync_copy(aux_hbm, aux_v)

        for t in range(NCH):
            rb = base + t * C
            pltpu.sync_copy(ids_hbm.at[pl.ds(rb, C)], raw_v)
            any_sp = jnp.zeros((_L,), jnp.int32)
            for g in range(C // _L):
                v = raw_v[pl.ds(g * _L, _L)]
                sp = (lax.shift_right_logical(v - 1, 31)
                      | lax.shift_right_logical(V - v, 31))
                any_sp = any_sp + sp
                idx_v[pl.ds(g * _L, _L)] = jnp.clip(v - 1, 0, V - 1)

            cps = [
                pltpu.async_copy(
                    main_hbm.at[idx_v.at[pl.ds(j * 128, 128)]],
                    buf_v.at[pl.ds(j * 128, 128)], sem)
                for j in range(NB)
            ]
            for cp in cps:
                cp.wait()

            nsp = any_sp[0]
            for q in range(1, _L):
                nsp = nsp + any_sp[q]

            @pl.when(nsp > 0)
            def _fixup():
                def fgrp(g, pos):
                    v = raw_v[pl.ds(g * _L, _L)]
                    kk = jnp.clip(v - V, 0, A - 1)

                    def fcol(c, cs):
                        m = plsc.bitcast(v - 1, jnp.uint32) > jnp.uint32(V - 1)
                        x = plsc.load_gather(aux_v, [kk, cs], mask=m)
                        plsc.store_scatter(buf_v, [pos, cs], x, mask=m)
                        return cs + 1

                    lax.fori_loop(0, D, fcol, jnp.zeros((_L,), jnp.int32))
                    return pos + _L

                lax.fori_loop(0, C // _L, fgrp, lax.iota(jnp.int32, _L))

            pltpu.sync_copy(buf_v, out_hbm.at[pl.ds(rb, C)])

    return k


def kernel(inputs, main_table, re_lut):
    B, H = inputs.shape
    V, D = main_table.shape
    A = re_lut.shape[0] + 2
    N = B * H
    ids = inputs.reshape(N).astype(jnp.int32)
    aux = jnp.concatenate(
        [jnp.zeros((2, D), jnp.float32), re_lut.astype(jnp.float32)], axis=0)
    k = _build(N, D, V, A, 2, 16)
    out = k(ids, main_table.astype(jnp.float32), aux)
    return out.reshape(B, H, D)
